# Initial kernel scaffold; baseline (speedup 1.0000x reference)
import jax
import jax.numpy as jnp
from jax.experimental import pallas as pl


def kernel(preds, labels, pad_mask):
    raise NotImplementedError("write your pallas kernel here")


# speedup vs baseline: 7.9431x; 7.9431x over previous
import jax
import jax.numpy as jnp
from jax import lax
from jax.experimental import pallas as pl
from jax.experimental.pallas import tpu as pltpu
from jax.experimental.pallas import tpu_sc as plsc

C = 128
LANES = 16
NSUB = 16
ROWS = 128
COLS = 128
RPW = ROWS // NSUB
VPR = COLS // LANES


def _wce_body(preds_hbm, lab_hbm, mask_hbm, out_hbm,
              lab_v, mask_v, idx_v, g_v, cntb, psumb, part_v, allp_v,
              out_v, shared, sem):
    w = lax.axis_index("s")
    r0 = w * RPW
    pltpu.sync_copy(lab_hbm.at[pl.ds(r0, RPW)], lab_v)
    pltpu.sync_copy(mask_hbm.at[pl.ds(r0, RPW)], mask_v)

    iota = lax.iota(jnp.int32, LANES)
    for r in range(RPW):
        row_base = (r0 + r) * COLS
        for k in range(VPR):
            sl = pl.ds(k * LANES, LANES)
            tok = row_base + k * LANES + iota
            idx_v[r, sl] = tok * C + lab_v[r, sl]

    copies = [pltpu.async_copy(preds_hbm.at[idx_v.at[r]], g_v.at[r], sem)
              for r in range(RPW)]

    zero = jnp.zeros((LANES,), jnp.float32)
    for i in range(C * LANES // LANES):
        cntb[pl.ds(i * LANES, LANES)] = zero
        psumb[pl.ds(i * LANES, LANES)] = zero

    for cp in copies:
        cp.wait()

    lane_off = iota * C
    for r in range(RPW):
        for k in range(VPR):
            sl = pl.ds(k * LANES, LANES)
            lab = lab_v[r, sl]
            m = mask_v[r, sl]
            g = g_v[r, sl]
            bidx = lane_off + lab
            plsc.addupdate_scatter(cntb, [bidx], m)
            plsc.addupdate_scatter(psumb, [bidx], g * m)

    for k in range(VPR):
        sl = pl.ds(k * LANES, LANES)
        ac = cntb[sl]
        ap = psumb[sl]
        for l in range(1, LANES):
            off = l * C + k * LANES
            ac = ac + cntb[pl.ds(off, LANES)]
            ap = ap + psumb[pl.ds(off, LANES)]
        part_v[sl] = ac
        part_v[pl.ds(C + k * LANES, LANES)] = ap

    pltpu.sync_copy(part_v, shared.at[w])
    plsc.subcore_barrier()

    @pl.when(w == 0)
    def _final():
        pltpu.sync_copy(shared, allp_v)
        cnt, ps = [], []
        for k in range(VPR):
            ac = allp_v[0, pl.ds(k * LANES, LANES)]
            ap = allp_v[0, pl.ds(C + k * LANES, LANES)]
            for t in range(1, NSUB):
                ac = ac + allp_v[t, pl.ds(k * LANES, LANES)]
                ap = ap + allp_v[t, pl.ds(C + k * LANES, LANES)]
            cnt.append(ac)
            ps.append(ap)
        mv = cnt[0]
        for k in range(1, VPR):
            mv = jnp.minimum(mv, cnt[k])
        mmin = jnp.min(mv)
        num = jnp.zeros((LANES,), jnp.float32)
        den = jnp.zeros((LANES,), jnp.float32)
        for k in range(VPR):
            wgt = mmin / (cnt[k] + 1e-8)
            num = num + wgt * ps[k]
            den = den + wgt * cnt[k]
        numv = jnp.full((LANES,), jnp.sum(num), jnp.float32)
        denv = jnp.full((LANES,), jnp.sum(den), jnp.float32)
        out_v[...] = -(numv / denv)
        pltpu.sync_copy(out_v, out_hbm)


def kernel(preds, labels, pad_mask):
    b, s, c = preds.shape
    n = b * s
    preds_f = preds.reshape(n * c)
    lab = labels.reshape(ROWS, COLS).astype(jnp.int32)
    mf = pad_mask.reshape(ROWS, COLS).astype(jnp.float32)
    mesh = plsc.VectorSubcoreMesh(
        core_axis_name="c", subcore_axis_name="s", num_cores=1)
    out = pl.kernel(
        _wce_body,
        out_type=jax.ShapeDtypeStruct((LANES,), jnp.float32),
        mesh=mesh,
        compiler_params=pltpu.CompilerParams(needs_layout_passes=False),
        scratch_types=[
            pltpu.VMEM((RPW, COLS), jnp.int32),
            pltpu.VMEM((RPW, COLS), jnp.float32),
            pltpu.VMEM((RPW, COLS), jnp.int32),
            pltpu.VMEM((RPW, COLS), jnp.float32),
            pltpu.VMEM((LANES * C,), jnp.float32),
            pltpu.VMEM((LANES * C,), jnp.float32),
            pltpu.VMEM((2 * C,), jnp.float32),
            pltpu.VMEM((NSUB, 2 * C), jnp.float32),
            pltpu.VMEM((LANES,), jnp.float32),
            pltpu.VMEM_SHARED((NSUB, 2 * C), jnp.float32),
            pltpu.SemaphoreType.DMA,
        ],
    )(preds_f, lab, mf)
    return out[0]


# speedup vs baseline: 8.2718x; 1.0414x over previous
import jax
import jax.numpy as jnp
from jax import lax
from jax.experimental import pallas as pl
from jax.experimental.pallas import tpu as pltpu
from jax.experimental.pallas import tpu_sc as plsc

C = 128
LANES = 16
NSUB = 16
NTOK = 16384
TPW = NTOK // NSUB
RPW = 8
COLS = TPW // RPW
VPR = COLS // LANES
NBIN = LANES * C
DEAD = NBIN
MROWS = NTOK // COLS


def _wce_body(preds_hbm, ml_hbm, out_hbm,
              ml_v, idx_v, bidx_v, g_v, cntb, psumb, part_v,
              allp_v, out_v, shared, sem, sem2):
    w = lax.axis_index("s")
    base = w * TPW
    pltpu.async_copy(ml_hbm.at[pl.ds(w * RPW, RPW)], ml_v, sem2).wait()

    iota = lax.iota(jnp.int32, LANES)
    lane_row = iota * C
    dead = DEAD + iota
    zerov = jnp.zeros((LANES,), jnp.float32)
    onev = jnp.ones((LANES,), jnp.float32)

    gcopies = []
    for r in range(RPW):
        rbase = (base + r * COLS) * C
        for k in range(VPR):
            sl = pl.ds(k * LANES, LANES)
            ml = ml_v[r, sl]
            live = lane_row + (ml & (C - 1))
            bidx_v[r, sl] = jnp.where(ml >= 256, live, dead)
            idx_v[r, sl] = rbase + k * (LANES * C) + live
        gcopies.append(pltpu.async_copy(preds_hbm.at[idx_v.at[r]], g_v.at[r], sem))

    for i in range(NBIN // LANES):
        cntb[pl.ds(i * LANES, LANES)] = zerov
        psumb[pl.ds(i * LANES, LANES)] = zerov

    for r in range(RPW):
        gcopies[r].wait()
        for k in range(VPR):
            sl = pl.ds(k * LANES, LANES)
            bidx = bidx_v[r, sl]
            g = g_v[r, sl]
            plsc.addupdate_scatter(cntb, [bidx], onev)
            plsc.addupdate_scatter(psumb, [bidx], g)

    for k in range(C // LANES):
        sl = pl.ds(k * LANES, LANES)
        ac = cntb[sl]
        ap = psumb[sl]
        for l in range(1, LANES):
            off = l * C + k * LANES
            ac = ac + cntb[pl.ds(off, LANES)]
            ap = ap + psumb[pl.ds(off, LANES)]
        part_v[sl] = ac
        part_v[pl.ds(C + k * LANES, LANES)] = ap

    pltpu.sync_copy(part_v, shared.at[w])
    plsc.subcore_barrier()

    @pl.when(w == 0)
    def _final():
        pltpu.sync_copy(shared, allp_v)
        cnt, ps = [], []
        for k in range(C // LANES):
            ac = allp_v[0, pl.ds(k * LANES, LANES)]
            ap = allp_v[0, pl.ds(C + k * LANES, LANES)]
            for t in range(1, NSUB):
                ac = ac + allp_v[t, pl.ds(k * LANES, LANES)]
                ap = ap + allp_v[t, pl.ds(C + k * LANES, LANES)]
            cnt.append(ac)
            ps.append(ap)
        mv = cnt[0]
        for k in range(1, C // LANES):
            mv = jnp.minimum(mv, cnt[k])
        mmin = jnp.min(mv)
        num = jnp.zeros((LANES,), jnp.float32)
        den = jnp.zeros((LANES,), jnp.float32)
        for k in range(C // LANES):
            wgt = mmin / (cnt[k] + 1e-8)
            num = num + wgt * ps[k]
            den = den + wgt * cnt[k]
        numv = jnp.full((LANES,), jnp.sum(num), jnp.float32)
        denv = jnp.full((LANES,), jnp.sum(den), jnp.float32)
        out_v[...] = -(numv / denv)
        pltpu.sync_copy(out_v, out_hbm)


def kernel(preds, labels, pad_mask):
    b, s, c = preds.shape
    preds_f = preds.reshape(b * s * c)
    ml = (labels.astype(jnp.int32)
          | (pad_mask.astype(jnp.int32) << 8)).reshape(MROWS, COLS)
    mesh = plsc.VectorSubcoreMesh(
        core_axis_name="c", subcore_axis_name="s", num_cores=1)
    out = pl.kernel(
        _wce_body,
        out_type=jax.ShapeDtypeStruct((LANES,), jnp.float32),
        mesh=mesh,
        compiler_params=pltpu.CompilerParams(needs_layout_passes=False),
        scratch_types=[
            pltpu.VMEM((RPW, COLS), jnp.int32),
            pltpu.VMEM((RPW, COLS), jnp.int32),
            pltpu.VMEM((RPW, COLS), jnp.int32),
            pltpu.VMEM((RPW, COLS), jnp.float32),
            pltpu.VMEM((NBIN + LANES,), jnp.float32),
            pltpu.VMEM((NBIN + LANES,), jnp.float32),
            pltpu.VMEM((2 * C,), jnp.float32),
            pltpu.VMEM((NSUB, 2 * C), jnp.float32),
            pltpu.VMEM((LANES,), jnp.float32),
            pltpu.VMEM_SHARED((NSUB, 2 * C), jnp.float32),
            pltpu.SemaphoreType.DMA,
            pltpu.SemaphoreType.DMA,
        ],
    )(preds_f, ml)
    return out[0]
